# baseline (speedup 1.0000x reference)
import functools

import jax
import jax.numpy as jnp
from jax import lax
from jax.experimental import pallas as pl
from jax.experimental.pallas import tpu as pltpu
from jax.experimental.pallas import tpu_sc as plsc

N = 10000
E = 320000
D = 128
B = 1024
NC, NS = 2, 16
NW = NC * NS
K = 80
NCH_N = N // K
EW = E // NW
NCH_E = EW // K
ESUB = E // NS
K2 = 80
NCH_S = ESUB // K2
NCH_N2 = N // K2
DH = D // 2
NB2 = 2
SST = (130, 120)
BW = B // NW
RT = 2000
GT = N // RT

_mesh = plsc.VectorSubcoreMesh(
    core_axis_name="c", subcore_axis_name="s", num_cores=NC, num_subcores=NS)

_f32 = jnp.float32


def _fill(ref, val, n16):
  for j in range(n16):
    ref[pl.ds(j * 16, 16)] = jnp.full((16,), val, _f32)


@functools.partial(
    pl.kernel,
    out_type=(
        jax.ShapeDtypeStruct((N, D), _f32),
        jax.ShapeDtypeStruct((N, D), _f32),
        jax.ShapeDtypeStruct((B, D), _f32),
        jax.ShapeDtypeStruct((NC * N,), _f32),
        jax.ShapeDtypeStruct((NC * N,), _f32),
    ),
    mesh=_mesh,
    scratch_types=[
        pltpu.VMEM((4, K), jnp.int32),
        pltpu.VMEM((4, K, D), _f32),
        pltpu.VMEM((BW,), jnp.int32),
        pltpu.VMEM((BW, D), _f32),
        pltpu.VMEM((NCH_E, K), jnp.int32),
        pltpu.VMEM((K,), _f32),
        pltpu.VMEM((K,), _f32),
        pltpu.VMEM_SHARED((N,), _f32),
        pltpu.VMEM_SHARED((N,), _f32),
        pltpu.SemaphoreType.DMA,
        pltpu.SemaphoreType.DMA,
    ],
)
def _sc_gather_deg(u_ids_h, v_ids_h, q_h, emb_h, dstu_h, dstv_h,
                   hu_h, hv_h, hq_h, degu_h, degv_h,
                   idx_v, rows_v, qidx_v, qrows_v, dstbuf, ones_v, line_v,
                   degu_s, degv_s, semg, semw):
  cid = lax.axis_index("c")
  sid = lax.axis_index("s")
  wid = sid * NC + cid

  _fill(line_v, 0.0, K // 16)
  for k in range(8):
    c = sid * 8 + k
    @pl.when(c < NCH_N)
    def _():
      off = pl.multiple_of(c * K, 8)
      pltpu.sync_copy(line_v, degu_s.at[pl.ds(off, K)])
      pltpu.sync_copy(line_v, degv_s.at[pl.ds(off, K)])

  _fill(ones_v, 1.0, K // 16)
  plsc.subcore_barrier()

  qoff = pl.multiple_of(wid * BW, 8)
  pltpu.sync_copy(q_h.at[pl.ds(qoff, BW)], qidx_v)
  dq = pltpu.async_copy(emb_h.at[qidx_v], qrows_v, semw)

  for ids_h, out_h in ((u_ids_h, hu_h), (v_ids_h, hv_h)):
    for k in range(4):
      c = wid + NW * k
      @pl.when(c < NCH_N)
      def _():
        off = pl.multiple_of(c * K, 8)
        pltpu.sync_copy(ids_h.at[pl.ds(off, K)], idx_v.at[k])
    gds = []
    for k in range(4):
      c = wid + NW * k
      @pl.when(c < NCH_N)
      def _():
        gds.append(pltpu.async_copy(emb_h.at[idx_v.at[k]], rows_v.at[k], semg))
    for k in range(4):
      c = wid + NW * k
      @pl.when(c < NCH_N)
      def _():
        gds.pop(0).wait()
    for k in range(4):
      c = wid + NW * k
      @pl.when(c < NCH_N)
      def _():
        off = pl.multiple_of(c * K, 8)
        pltpu.sync_copy(rows_v.at[k], out_h.at[pl.ds(off, K)])

  dq.wait()
  pltpu.sync_copy(qrows_v, hq_h.at[pl.ds(qoff, BW)])

  for dst_h, deg_s in ((dstu_h, degu_s), (dstv_h, degv_s)):
    pltpu.sync_copy(dst_h.at[wid], dstbuf)

    @pl.loop(0, NCH_E // 5)
    def _(t):
      ds_ = [pltpu.async_copy(ones_v, deg_s.at[dstbuf.at[t * 5 + b]],
                              semw, add=True) for b in range(5)]
      for d in ds_:
        d.wait()

  plsc.subcore_barrier()

  for k in range(8):
    c = sid * 8 + k
    @pl.when(c < NCH_N)
    def _():
      off = pl.multiple_of(c * K, 8)
      foff = pl.multiple_of(cid * N + c * K, 8)
      pltpu.sync_copy(degu_s.at[pl.ds(off, K)], line_v)
      pltpu.sync_copy(line_v, degu_h.at[pl.ds(foff, K)])
      pltpu.sync_copy(degv_s.at[pl.ds(off, K)], line_v)
      pltpu.sync_copy(line_v, degv_h.at[pl.ds(foff, K)])


@functools.partial(
    pl.kernel,
    out_type=(
        jax.ShapeDtypeStruct((NC, N, DH), _f32),
        jax.ShapeDtypeStruct((N,), _f32),
    ),
    mesh=_mesh,
    scratch_types=[
        pltpu.VMEM((SST[0], K2), jnp.int32),
        pltpu.VMEM((SST[0], K2), jnp.int32),
        pltpu.VMEM((NB2, K2, DH), _f32),
        pltpu.VMEM((NB2, K2), _f32),
        pltpu.VMEM_SHARED((N, DH), _f32),
        pltpu.VMEM_SHARED((N,), _f32),
        pltpu.VMEM_SHARED((N, DH), _f32),
        pltpu.VMEM_SHARED((N,), _f32),
        pltpu.SemaphoreType.DMA,
        pltpu.SemaphoreType.DMA,
        pltpu.SemaphoreType.DMA,
        pltpu.SemaphoreType.DMA,
        pltpu.SemaphoreType.DMA,
        pltpu.SemaphoreType.DMA,
        pltpu.SemaphoreType.DMA,
    ],
    compiler_params=pltpu.CompilerParams(use_tc_tiling_on_sc=False),
)
def _sc_edge_agg(yl_h, yr_h, dinv_h, src_h, dst_h,
                 acc_h, s_h,
                 srcbuf, dstbuf, rbuf, vals_v,
                 acc_s, s_s, y_s, dinv_s,
                 semga, semgb, semsa, semsb, semsg, semsh, semss):
  cid = lax.axis_index("c")
  sid = lax.axis_index("s")

  @pl.loop(0, K2)
  def _(i):
    for j in range(DH // 16):
      rbuf[0, i, pl.ds(j * 16, 16)] = jnp.zeros((16,), _f32)

  for j in range(K2 // 16):
    vals_v[0, pl.ds(j * 16, 16)] = jnp.zeros((16,), _f32)

  for k in range(8):
    c = sid * 8 + k
    @pl.when(c < NCH_N2)
    def _():
      off = pl.multiple_of(c * K2, 8)
      pltpu.sync_copy(rbuf.at[0], acc_s.at[pl.ds(off, K2)])

      @pl.when(cid == 0)
      def _():
        pltpu.sync_copy(vals_v.at[0], s_s.at[pl.ds(off, K2)])
        pltpu.sync_copy(yl_h.at[pl.ds(off, K2)], y_s.at[pl.ds(off, K2)])
        pltpu.sync_copy(dinv_h.at[pl.ds(off, K2)], vals_v.at[1])
        pltpu.sync_copy(vals_v.at[1], dinv_s.at[pl.ds(off, K2)])

      @pl.when(cid == 1)
      def _():
        pltpu.sync_copy(yr_h.at[pl.ds(off, K2)], y_s.at[pl.ds(off, K2)])

  plsc.subcore_barrier()

  H2 = NB2 // 2

  def _pipe(do_s):
    lo = 0
    for nch in SST:
      pltpu.sync_copy(src_h.at[sid, pl.ds(lo, nch)],
                      srcbuf.at[pl.ds(0, nch)])
      pltpu.sync_copy(dst_h.at[sid, pl.ds(lo, nch)],
                      dstbuf.at[pl.ds(0, nch)])

      @pl.loop(0, nch // NB2)
      def _(t):
        loc = t * NB2
        ga = [pltpu.async_copy(y_s.at[srcbuf.at[loc + b]], rbuf.at[b], semga)
              for b in range(H2)]
        gsa = [pltpu.async_copy(dinv_s.at[dstbuf.at[loc + b]],
                                vals_v.at[b], semsg)
               for b in range(H2)] if do_s else []
        gb = [pltpu.async_copy(y_s.at[srcbuf.at[loc + H2 + b]],
                               rbuf.at[H2 + b], semgb)
              for b in range(H2)]
        gsb = [pltpu.async_copy(dinv_s.at[dstbuf.at[loc + H2 + b]],
                                vals_v.at[H2 + b], semsh)
               for b in range(H2)] if do_s else []
        for d in ga:
          d.wait()
        sa = [pltpu.async_copy(rbuf.at[b], acc_s.at[dstbuf.at[loc + b]],
                               semsa, add=True)
              for b in range(H2)]
        for d in gsa:
          d.wait()
        ssa = [pltpu.async_copy(vals_v.at[b],
                                s_s.at[srcbuf.at[loc + b]], semss, add=True)
               for b in range(H2)] if do_s else []
        for d in gb:
          d.wait()
        sb = [pltpu.async_copy(rbuf.at[H2 + b],
                               acc_s.at[dstbuf.at[loc + H2 + b]],
                               semsb, add=True)
              for b in range(H2)]
        for d in gsb:
          d.wait()
        ssb = [pltpu.async_copy(vals_v.at[H2 + b],
                                s_s.at[srcbuf.at[loc + H2 + b]], semss,
                                add=True)
               for b in range(H2)] if do_s else []
        for d in sa + sb + ssa + ssb:
          d.wait()

      lo += nch

  @pl.when(cid == 0)
  def _():
    _pipe(True)

  @pl.when(cid == 1)
  def _():
    _pipe(False)

  plsc.subcore_barrier()

  for k in range(8):
    c = sid * 8 + k
    @pl.when(c < NCH_N2)
    def _():
      off = pl.multiple_of(c * K2, 8)
      pltpu.sync_copy(acc_s.at[pl.ds(off, K2)], rbuf.at[0])
      pltpu.sync_copy(rbuf.at[0], acc_h.at[cid, pl.ds(off, K2)])

      @pl.when(cid == 0)
      def _():
        pltpu.sync_copy(s_s.at[pl.ds(off, K2)], vals_v.at[0])
        pltpu.sync_copy(vals_v.at[0], s_h.at[pl.ds(off, K2)])


def _tc_prep(h, deg_t, W1):
  def body(h_ref, deg_ref, w_ref, yl_ref, yr_ref, dinv_ref):
    deg = deg_ref[:, 0:1] + deg_ref[:, 1:2] + 1.0
    dinv = lax.rsqrt(jnp.maximum(deg, 1.0))
    dinv_ref[...] = dinv
    y = jnp.dot(dinv * h_ref[...], w_ref[...],
                preferred_element_type=_f32,
                precision=lax.Precision.HIGHEST)
    yl_ref[...] = y[:, :DH]
    yr_ref[...] = y[:, DH:]

  return pl.pallas_call(
      body,
      grid=(GT,),
      in_specs=[
          pl.BlockSpec((RT, D), lambda i: (i, 0)),
          pl.BlockSpec((RT, NC), lambda i: (i, 0)),
          pl.BlockSpec((D, D), lambda i: (0, 0)),
      ],
      out_specs=[
          pl.BlockSpec((RT, DH), lambda i: (i, 0)),
          pl.BlockSpec((RT, DH), lambda i: (i, 0)),
          pl.BlockSpec((RT, 1), lambda i: (i, 0)),
      ],
      out_shape=(
          jax.ShapeDtypeStruct((N, DH), _f32),
          jax.ShapeDtypeStruct((N, DH), _f32),
          jax.ShapeDtypeStruct((N, 1), _f32),
      ),
  )(h, deg_t, W1)


def _tc_readout(acc_p, yl, yr, dinv, s1, b1, W2, b2):
  def body(acc_ref, yl_ref, yr_ref, dinv_ref, s_ref, b1_ref, w2_ref, b2_ref,
           hg_ref, t_ref):
    i = pl.program_id(0)

    @pl.when(i == 0)
    def _():
      t_ref[...] = jnp.zeros_like(t_ref)

    dinv = dinv_ref[...]
    w = dinv * (dinv + s_ref[...]) * (1.0 / N)
    for half, acc_half, y_ref in ((0, acc_ref[0], yl_ref),
                                  (1, acc_ref[1], yr_ref)):
      acc = acc_half + y_ref[...]
      pre = dinv * acc + b1_ref[:, half * DH:(half + 1) * DH]
      h1 = jnp.where(pre >= 0, pre, 0.01 * pre)
      t_ref[:, half * DH:(half + 1) * DH] += jnp.sum(
          w * h1, axis=0, keepdims=True)

    @pl.when(i == GT - 1)
    def _():
      hg_ref[...] = jnp.dot(t_ref[...], w2_ref[...],
                            preferred_element_type=_f32,
                            precision=lax.Precision.HIGHEST) + b2_ref[...]

  return pl.pallas_call(
      body,
      grid=(GT,),
      in_specs=[
          pl.BlockSpec((NC, RT, DH), lambda i: (0, i, 0)),
          pl.BlockSpec((RT, DH), lambda i: (i, 0)),
          pl.BlockSpec((RT, DH), lambda i: (i, 0)),
          pl.BlockSpec((RT, 1), lambda i: (i, 0)),
          pl.BlockSpec((RT, 1), lambda i: (i, 0)),
          pl.BlockSpec((1, D), lambda i: (0, 0)),
          pl.BlockSpec((D, D), lambda i: (0, 0)),
          pl.BlockSpec((1, D), lambda i: (0, 0)),
      ],
      out_specs=pl.BlockSpec((1, D), lambda i: (0, 0)),
      out_shape=jax.ShapeDtypeStruct((1, D), _f32),
      scratch_shapes=[pltpu.VMEM((1, D), _f32)],
  )(acc_p, yl, yr, dinv, s1, b1, W2, b2)


def _tc_match(hq, hgu, hgv, Wm_u, Wm_v):
  def body(hq_ref, hgu_ref, hgv_ref, wu_ref, wv_ref, out_ref):
    dn = (((1,), (1,)), ((), ()))
    zu = lax.dot_general(hgu_ref[...], wu_ref[...], dn,
                         preferred_element_type=_f32,
                         precision=lax.Precision.HIGHEST)
    zv = lax.dot_general(hgv_ref[...], wv_ref[...], dn,
                         preferred_element_type=_f32,
                         precision=lax.Precision.HIGHEST)
    z = zu + zv
    out_ref[...] = lax.dot_general(hq_ref[...], z, dn,
                                   preferred_element_type=_f32,
                                   precision=lax.Precision.HIGHEST)

  return pl.pallas_call(
      body,
      out_shape=jax.ShapeDtypeStruct((B, 1), _f32),
  )(hq, hgu, hgv, Wm_u, Wm_v)


def kernel(u_node_ids, u_edge_index, v_node_ids, v_edge_index, q,
           embedding, pW1, pb1, pW2, pb2, cW1, cb1, cW2, cb2, Wm_u, Wm_v):
  u_ids = u_node_ids.astype(jnp.int32)
  v_ids = v_node_ids.astype(jnp.int32)
  qi = q.astype(jnp.int32)
  ue = u_edge_index.astype(jnp.int32)
  ve = v_edge_index.astype(jnp.int32)
  dstw_u = ue[1].reshape(NW, NCH_E, K)
  dstw_v = ve[1].reshape(NW, NCH_E, K)
  src_u = ue[0].reshape(NS, NCH_S, K2)
  dst_u = ue[1].reshape(NS, NCH_S, K2)
  src_v = ve[0].reshape(NS, NCH_S, K2)
  dst_v = ve[1].reshape(NS, NCH_S, K2)

  hu, hv, hq, degu_p, degv_p = _sc_gather_deg(
      u_ids, v_ids, qi, embedding, dstw_u, dstw_v)

  yl_u, yr_u, dinv_u = _tc_prep(hu, degu_p.reshape(NC, N).T, pW1)
  yl_v, yr_v, dinv_v = _tc_prep(hv, degv_p.reshape(NC, N).T, cW1)

  acc_u_p, s_u = _sc_edge_agg(yl_u, yr_u, dinv_u.reshape(N), src_u, dst_u)
  acc_v_p, s_v = _sc_edge_agg(yl_v, yr_v, dinv_v.reshape(N), src_v, dst_v)

  hgu = _tc_readout(acc_u_p, yl_u, yr_u, dinv_u, s_u.reshape(N, 1),
                    pb1.reshape(1, D), pW2, pb2.reshape(1, D))
  hgv = _tc_readout(acc_v_p, yl_v, yr_v, dinv_v, s_v.reshape(N, 1),
                    cb1.reshape(1, D), cW2, cb2.reshape(1, D))

  scores = _tc_match(hq, hgu, hgv, Wm_u, Wm_v)
  return scores.reshape(B)


# speedup vs baseline: 1.0060x; 1.0060x over previous
import functools

import jax
import jax.numpy as jnp
from jax import lax
from jax.experimental import pallas as pl
from jax.experimental.pallas import tpu as pltpu
from jax.experimental.pallas import tpu_sc as plsc

N = 10000
E = 320000
D = 128
B = 1024
NC, NS = 2, 16
NW = NC * NS
K = 80
NCH_N = N // K
EW = E // NW
NCH_E = EW // K
ESUB = E // NS
K2 = 80
NCH_S = ESUB // K2
NCH_N2 = N // K2
DH = D // 2
NB2 = 4
SST = ((128, 4), (120, 4), (2, 2))
BW = B // NW
RT = 2000
GT = N // RT

_mesh = plsc.VectorSubcoreMesh(
    core_axis_name="c", subcore_axis_name="s", num_cores=NC, num_subcores=NS)

_f32 = jnp.float32


def _fill(ref, val, n16):
  for j in range(n16):
    ref[pl.ds(j * 16, 16)] = jnp.full((16,), val, _f32)


@functools.partial(
    pl.kernel,
    out_type=(
        jax.ShapeDtypeStruct((N, D), _f32),
        jax.ShapeDtypeStruct((N, D), _f32),
        jax.ShapeDtypeStruct((B, D), _f32),
        jax.ShapeDtypeStruct((NC * N,), _f32),
        jax.ShapeDtypeStruct((NC * N,), _f32),
    ),
    mesh=_mesh,
    scratch_types=[
        pltpu.VMEM((4, K), jnp.int32),
        pltpu.VMEM((4, K, D), _f32),
        pltpu.VMEM((BW,), jnp.int32),
        pltpu.VMEM((BW, D), _f32),
        pltpu.VMEM((NCH_E, K), jnp.int32),
        pltpu.VMEM((K,), _f32),
        pltpu.VMEM((K,), _f32),
        pltpu.VMEM_SHARED((N,), _f32),
        pltpu.VMEM_SHARED((N,), _f32),
        pltpu.SemaphoreType.DMA,
        pltpu.SemaphoreType.DMA,
    ],
)
def _sc_gather_deg(u_ids_h, v_ids_h, q_h, emb_h, dstu_h, dstv_h,
                   hu_h, hv_h, hq_h, degu_h, degv_h,
                   idx_v, rows_v, qidx_v, qrows_v, dstbuf, ones_v, line_v,
                   degu_s, degv_s, semg, semw):
  cid = lax.axis_index("c")
  sid = lax.axis_index("s")
  wid = sid * NC + cid

  _fill(line_v, 0.0, K // 16)
  for k in range(8):
    c = sid * 8 + k
    @pl.when(c < NCH_N)
    def _():
      off = pl.multiple_of(c * K, 8)
      pltpu.sync_copy(line_v, degu_s.at[pl.ds(off, K)])
      pltpu.sync_copy(line_v, degv_s.at[pl.ds(off, K)])

  _fill(ones_v, 1.0, K // 16)
  plsc.subcore_barrier()

  qoff = pl.multiple_of(wid * BW, 8)
  pltpu.sync_copy(q_h.at[pl.ds(qoff, BW)], qidx_v)
  dq = pltpu.async_copy(emb_h.at[qidx_v], qrows_v, semw)

  for ids_h, out_h in ((u_ids_h, hu_h), (v_ids_h, hv_h)):
    for k in range(4):
      c = wid + NW * k
      @pl.when(c < NCH_N)
      def _():
        off = pl.multiple_of(c * K, 8)
        pltpu.sync_copy(ids_h.at[pl.ds(off, K)], idx_v.at[k])
    gds = []
    for k in range(4):
      c = wid + NW * k
      @pl.when(c < NCH_N)
      def _():
        gds.append(pltpu.async_copy(emb_h.at[idx_v.at[k]], rows_v.at[k], semg))
    for k in range(4):
      c = wid + NW * k
      @pl.when(c < NCH_N)
      def _():
        gds.pop(0).wait()
    for k in range(4):
      c = wid + NW * k
      @pl.when(c < NCH_N)
      def _():
        off = pl.multiple_of(c * K, 8)
        pltpu.sync_copy(rows_v.at[k], out_h.at[pl.ds(off, K)])

  dq.wait()
  pltpu.sync_copy(qrows_v, hq_h.at[pl.ds(qoff, BW)])

  for dst_h, deg_s in ((dstu_h, degu_s), (dstv_h, degv_s)):
    pltpu.sync_copy(dst_h.at[wid], dstbuf)

    @pl.loop(0, NCH_E // 5)
    def _(t):
      ds_ = [pltpu.async_copy(ones_v, deg_s.at[dstbuf.at[t * 5 + b]],
                              semw, add=True) for b in range(5)]
      for d in ds_:
        d.wait()

  plsc.subcore_barrier()

  for k in range(8):
    c = sid * 8 + k
    @pl.when(c < NCH_N)
    def _():
      off = pl.multiple_of(c * K, 8)
      foff = pl.multiple_of(cid * N + c * K, 8)
      pltpu.sync_copy(degu_s.at[pl.ds(off, K)], line_v)
      pltpu.sync_copy(line_v, degu_h.at[pl.ds(foff, K)])
      pltpu.sync_copy(degv_s.at[pl.ds(off, K)], line_v)
      pltpu.sync_copy(line_v, degv_h.at[pl.ds(foff, K)])


@functools.partial(
    pl.kernel,
    out_type=(
        jax.ShapeDtypeStruct((NC, N, DH), _f32),
        jax.ShapeDtypeStruct((N,), _f32),
    ),
    mesh=_mesh,
    scratch_types=[
        pltpu.VMEM((SST[0][0], K2), jnp.int32),
        pltpu.VMEM((SST[0][0], K2), jnp.int32),
        pltpu.VMEM((NB2, K2, DH), _f32),
        pltpu.VMEM((NB2, K2), _f32),
        pltpu.VMEM_SHARED((N, DH), _f32),
        pltpu.VMEM_SHARED((N,), _f32),
        pltpu.VMEM_SHARED((N, DH), _f32),
        pltpu.VMEM_SHARED((N,), _f32),
        pltpu.SemaphoreType.DMA,
        pltpu.SemaphoreType.DMA,
        pltpu.SemaphoreType.DMA,
        pltpu.SemaphoreType.DMA,
        pltpu.SemaphoreType.DMA,
        pltpu.SemaphoreType.DMA,
        pltpu.SemaphoreType.DMA,
    ],
    compiler_params=pltpu.CompilerParams(use_tc_tiling_on_sc=False),
)
def _sc_edge_agg(yl_h, yr_h, dinv_h, src_h, dst_h,
                 acc_h, s_h,
                 srcbuf, dstbuf, rbuf, vals_v,
                 acc_s, s_s, y_s, dinv_s,
                 semga, semgb, semsa, semsb, semsg, semsh, semss):
  cid = lax.axis_index("c")
  sid = lax.axis_index("s")

  @pl.loop(0, K2)
  def _(i):
    for j in range(DH // 16):
      rbuf[0, i, pl.ds(j * 16, 16)] = jnp.zeros((16,), _f32)

  for j in range(K2 // 16):
    vals_v[0, pl.ds(j * 16, 16)] = jnp.zeros((16,), _f32)

  for k in range(8):
    c = sid * 8 + k
    @pl.when(c < NCH_N2)
    def _():
      off = pl.multiple_of(c * K2, 8)
      pltpu.sync_copy(rbuf.at[0], acc_s.at[pl.ds(off, K2)])

      @pl.when(cid == 0)
      def _():
        pltpu.sync_copy(vals_v.at[0], s_s.at[pl.ds(off, K2)])
        pltpu.sync_copy(yl_h.at[pl.ds(off, K2)], y_s.at[pl.ds(off, K2)])
        pltpu.sync_copy(dinv_h.at[pl.ds(off, K2)], vals_v.at[1])
        pltpu.sync_copy(vals_v.at[1], dinv_s.at[pl.ds(off, K2)])

      @pl.when(cid == 1)
      def _():
        pltpu.sync_copy(yr_h.at[pl.ds(off, K2)], y_s.at[pl.ds(off, K2)])

  plsc.subcore_barrier()

  def _pipe(do_s):
    lo = 0
    for nch, nb in SST:
      H2 = nb // 2
      pltpu.sync_copy(src_h.at[sid, pl.ds(lo, nch)],
                      srcbuf.at[pl.ds(0, nch)])
      pltpu.sync_copy(dst_h.at[sid, pl.ds(lo, nch)],
                      dstbuf.at[pl.ds(0, nch)])

      @pl.loop(0, nch // nb)
      def _(t):
        loc = t * nb
        ga = [pltpu.async_copy(y_s.at[srcbuf.at[loc + b]], rbuf.at[b], semga)
              for b in range(H2)]
        gsa = [pltpu.async_copy(dinv_s.at[dstbuf.at[loc + b]],
                                vals_v.at[b], semsg)
               for b in range(H2)] if do_s else []
        gb = [pltpu.async_copy(y_s.at[srcbuf.at[loc + H2 + b]],
                               rbuf.at[H2 + b], semgb)
              for b in range(H2)]
        gsb = [pltpu.async_copy(dinv_s.at[dstbuf.at[loc + H2 + b]],
                                vals_v.at[H2 + b], semsh)
               for b in range(H2)] if do_s else []
        for d in ga:
          d.wait()
        sa = [pltpu.async_copy(rbuf.at[b], acc_s.at[dstbuf.at[loc + b]],
                               semsa, add=True)
              for b in range(H2)]
        for d in gsa:
          d.wait()
        ssa = [pltpu.async_copy(vals_v.at[b],
                                s_s.at[srcbuf.at[loc + b]], semss, add=True)
               for b in range(H2)] if do_s else []
        for d in gb:
          d.wait()
        sb = [pltpu.async_copy(rbuf.at[H2 + b],
                               acc_s.at[dstbuf.at[loc + H2 + b]],
                               semsb, add=True)
              for b in range(H2)]
        for d in gsb:
          d.wait()
        ssb = [pltpu.async_copy(vals_v.at[H2 + b],
                                s_s.at[srcbuf.at[loc + H2 + b]], semss,
                                add=True)
               for b in range(H2)] if do_s else []
        for d in sa + sb + ssa + ssb:
          d.wait()

      lo += nch

  @pl.when(cid == 0)
  def _():
    _pipe(True)

  @pl.when(cid == 1)
  def _():
    _pipe(False)

  plsc.subcore_barrier()

  for k in range(8):
    c = sid * 8 + k
    @pl.when(c < NCH_N2)
    def _():
      off = pl.multiple_of(c * K2, 8)
      pltpu.sync_copy(acc_s.at[pl.ds(off, K2)], rbuf.at[0])
      pltpu.sync_copy(rbuf.at[0], acc_h.at[cid, pl.ds(off, K2)])

      @pl.when(cid == 0)
      def _():
        pltpu.sync_copy(s_s.at[pl.ds(off, K2)], vals_v.at[0])
        pltpu.sync_copy(vals_v.at[0], s_h.at[pl.ds(off, K2)])


def _tc_prep(h, deg_t, W1):
  def body(h_ref, deg_ref, w_ref, yl_ref, yr_ref, dinv_ref):
    deg = deg_ref[:, 0:1] + deg_ref[:, 1:2] + 1.0
    dinv = lax.rsqrt(jnp.maximum(deg, 1.0))
    dinv_ref[...] = dinv
    y = jnp.dot(dinv * h_ref[...], w_ref[...],
                preferred_element_type=_f32,
                precision=lax.Precision.HIGHEST)
    yl_ref[...] = y[:, :DH]
    yr_ref[...] = y[:, DH:]

  return pl.pallas_call(
      body,
      grid=(GT,),
      in_specs=[
          pl.BlockSpec((RT, D), lambda i: (i, 0)),
          pl.BlockSpec((RT, NC), lambda i: (i, 0)),
          pl.BlockSpec((D, D), lambda i: (0, 0)),
      ],
      out_specs=[
          pl.BlockSpec((RT, DH), lambda i: (i, 0)),
          pl.BlockSpec((RT, DH), lambda i: (i, 0)),
          pl.BlockSpec((RT, 1), lambda i: (i, 0)),
      ],
      out_shape=(
          jax.ShapeDtypeStruct((N, DH), _f32),
          jax.ShapeDtypeStruct((N, DH), _f32),
          jax.ShapeDtypeStruct((N, 1), _f32),
      ),
  )(h, deg_t, W1)


def _tc_readout(acc_p, yl, yr, dinv, s1, b1, W2, b2):
  def body(acc_ref, yl_ref, yr_ref, dinv_ref, s_ref, b1_ref, w2_ref, b2_ref,
           hg_ref, t_ref):
    i = pl.program_id(0)

    @pl.when(i == 0)
    def _():
      t_ref[...] = jnp.zeros_like(t_ref)

    dinv = dinv_ref[...]
    w = dinv * (dinv + s_ref[...]) * (1.0 / N)
    for half, acc_half, y_ref in ((0, acc_ref[0], yl_ref),
                                  (1, acc_ref[1], yr_ref)):
      acc = acc_half + y_ref[...]
      pre = dinv * acc + b1_ref[:, half * DH:(half + 1) * DH]
      h1 = jnp.where(pre >= 0, pre, 0.01 * pre)
      t_ref[:, half * DH:(half + 1) * DH] += jnp.sum(
          w * h1, axis=0, keepdims=True)

    @pl.when(i == GT - 1)
    def _():
      hg_ref[...] = jnp.dot(t_ref[...], w2_ref[...],
                            preferred_element_type=_f32,
                            precision=lax.Precision.HIGHEST) + b2_ref[...]

  return pl.pallas_call(
      body,
      grid=(GT,),
      in_specs=[
          pl.BlockSpec((NC, RT, DH), lambda i: (0, i, 0)),
          pl.BlockSpec((RT, DH), lambda i: (i, 0)),
          pl.BlockSpec((RT, DH), lambda i: (i, 0)),
          pl.BlockSpec((RT, 1), lambda i: (i, 0)),
          pl.BlockSpec((RT, 1), lambda i: (i, 0)),
          pl.BlockSpec((1, D), lambda i: (0, 0)),
          pl.BlockSpec((D, D), lambda i: (0, 0)),
          pl.BlockSpec((1, D), lambda i: (0, 0)),
      ],
      out_specs=pl.BlockSpec((1, D), lambda i: (0, 0)),
      out_shape=jax.ShapeDtypeStruct((1, D), _f32),
      scratch_shapes=[pltpu.VMEM((1, D), _f32)],
  )(acc_p, yl, yr, dinv, s1, b1, W2, b2)


def _tc_match(hq, hgu, hgv, Wm_u, Wm_v):
  def body(hq_ref, hgu_ref, hgv_ref, wu_ref, wv_ref, out_ref):
    dn = (((1,), (1,)), ((), ()))
    zu = lax.dot_general(hgu_ref[...], wu_ref[...], dn,
                         preferred_element_type=_f32,
                         precision=lax.Precision.HIGHEST)
    zv = lax.dot_general(hgv_ref[...], wv_ref[...], dn,
                         preferred_element_type=_f32,
                         precision=lax.Precision.HIGHEST)
    z = zu + zv
    out_ref[...] = lax.dot_general(hq_ref[...], z, dn,
                                   preferred_element_type=_f32,
                                   precision=lax.Precision.HIGHEST)

  return pl.pallas_call(
      body,
      out_shape=jax.ShapeDtypeStruct((B, 1), _f32),
  )(hq, hgu, hgv, Wm_u, Wm_v)


def kernel(u_node_ids, u_edge_index, v_node_ids, v_edge_index, q,
           embedding, pW1, pb1, pW2, pb2, cW1, cb1, cW2, cb2, Wm_u, Wm_v):
  u_ids = u_node_ids.astype(jnp.int32)
  v_ids = v_node_ids.astype(jnp.int32)
  qi = q.astype(jnp.int32)
  ue = u_edge_index.astype(jnp.int32)
  ve = v_edge_index.astype(jnp.int32)
  dstw_u = ue[1].reshape(NW, NCH_E, K)
  dstw_v = ve[1].reshape(NW, NCH_E, K)
  src_u = ue[0].reshape(NS, NCH_S, K2)
  dst_u = ue[1].reshape(NS, NCH_S, K2)
  src_v = ve[0].reshape(NS, NCH_S, K2)
  dst_v = ve[1].reshape(NS, NCH_S, K2)

  hu, hv, hq, degu_p, degv_p = _sc_gather_deg(
      u_ids, v_ids, qi, embedding, dstw_u, dstw_v)

  yl_u, yr_u, dinv_u = _tc_prep(hu, degu_p.reshape(NC, N).T, pW1)
  yl_v, yr_v, dinv_v = _tc_prep(hv, degv_p.reshape(NC, N).T, cW1)

  acc_u_p, s_u = _sc_edge_agg(yl_u, yr_u, dinv_u.reshape(N), src_u, dst_u)
  acc_v_p, s_v = _sc_edge_agg(yl_v, yr_v, dinv_v.reshape(N), src_v, dst_v)

  hgu = _tc_readout(acc_u_p, yl_u, yr_u, dinv_u, s_u.reshape(N, 1),
                    pb1.reshape(1, D), pW2, pb2.reshape(1, D))
  hgv = _tc_readout(acc_v_p, yl_v, yr_v, dinv_v, s_v.reshape(N, 1),
                    cb1.reshape(1, D), cW2, cb2.reshape(1, D))

  scores = _tc_match(hq, hgu, hgv, Wm_u, Wm_v)
  return scores.reshape(B)


# speedup vs baseline: 1.2157x; 1.2084x over previous
import functools

import jax
import jax.numpy as jnp
from jax import lax
from jax.experimental import pallas as pl
from jax.experimental.pallas import tpu as pltpu
from jax.experimental.pallas import tpu_sc as plsc

N = 10000
E = 320000
D = 128
B = 1024
NC, NS = 2, 16
NW = NC * NS
K = 80
NCH_N = N // K
EW = E // NW
NCH_E = EW // K
ESUB = E // NS
K2 = 80
NCH_S = ESUB // K2
NCH_N2 = N // K2
DH = D // 2
NB2 = 10
SST = ((130, 10), (120, 10))
BW = B // NW
RT = 2000
GT = N // RT

_mesh = plsc.VectorSubcoreMesh(
    core_axis_name="c", subcore_axis_name="s", num_cores=NC, num_subcores=NS)

_f32 = jnp.float32


def _fill(ref, val, n16):
  for j in range(n16):
    ref[pl.ds(j * 16, 16)] = jnp.full((16,), val, _f32)


@functools.partial(
    pl.kernel,
    out_type=(
        jax.ShapeDtypeStruct((N, D), _f32),
        jax.ShapeDtypeStruct((N, D), _f32),
        jax.ShapeDtypeStruct((B, D), _f32),
        jax.ShapeDtypeStruct((NC * N,), _f32),
        jax.ShapeDtypeStruct((NC * N,), _f32),
    ),
    mesh=_mesh,
    scratch_types=[
        pltpu.VMEM((4, K), jnp.int32),
        pltpu.VMEM((4, K, D), _f32),
        pltpu.VMEM((BW,), jnp.int32),
        pltpu.VMEM((BW, D), _f32),
        pltpu.VMEM((NCH_E, K), jnp.int32),
        pltpu.VMEM((K,), _f32),
        pltpu.VMEM((K,), _f32),
        pltpu.VMEM_SHARED((N,), _f32),
        pltpu.VMEM_SHARED((N,), _f32),
        pltpu.SemaphoreType.DMA,
        pltpu.SemaphoreType.DMA,
    ],
)
def _sc_gather_deg(u_ids_h, v_ids_h, q_h, emb_h, dstu_h, dstv_h,
                   hu_h, hv_h, hq_h, degu_h, degv_h,
                   idx_v, rows_v, qidx_v, qrows_v, dstbuf, ones_v, line_v,
                   degu_s, degv_s, semg, semw):
  cid = lax.axis_index("c")
  sid = lax.axis_index("s")
  wid = sid * NC + cid

  _fill(line_v, 0.0, K // 16)
  for k in range(8):
    c = sid * 8 + k
    @pl.when(c < NCH_N)
    def _():
      off = pl.multiple_of(c * K, 8)
      pltpu.sync_copy(line_v, degu_s.at[pl.ds(off, K)])
      pltpu.sync_copy(line_v, degv_s.at[pl.ds(off, K)])

  _fill(ones_v, 1.0, K // 16)
  plsc.subcore_barrier()

  qoff = pl.multiple_of(wid * BW, 8)
  pltpu.sync_copy(q_h.at[pl.ds(qoff, BW)], qidx_v)
  dq = pltpu.async_copy(emb_h.at[qidx_v], qrows_v, semw)

  for ids_h, out_h in ((u_ids_h, hu_h), (v_ids_h, hv_h)):
    for k in range(4):
      c = wid + NW * k
      @pl.when(c < NCH_N)
      def _():
        off = pl.multiple_of(c * K, 8)
        pltpu.sync_copy(ids_h.at[pl.ds(off, K)], idx_v.at[k])
    gds = []
    for k in range(4):
      c = wid + NW * k
      @pl.when(c < NCH_N)
      def _():
        gds.append(pltpu.async_copy(emb_h.at[idx_v.at[k]], rows_v.at[k], semg))
    for k in range(4):
      c = wid + NW * k
      @pl.when(c < NCH_N)
      def _():
        gds.pop(0).wait()
    for k in range(4):
      c = wid + NW * k
      @pl.when(c < NCH_N)
      def _():
        off = pl.multiple_of(c * K, 8)
        pltpu.sync_copy(rows_v.at[k], out_h.at[pl.ds(off, K)])

  dq.wait()
  pltpu.sync_copy(qrows_v, hq_h.at[pl.ds(qoff, BW)])

  for dst_h, deg_s in ((dstu_h, degu_s), (dstv_h, degv_s)):
    pltpu.sync_copy(dst_h.at[wid], dstbuf)

    @pl.loop(0, NCH_E // 5)
    def _(t):
      ds_ = [pltpu.async_copy(ones_v, deg_s.at[dstbuf.at[t * 5 + b]],
                              semw, add=True) for b in range(5)]
      for d in ds_:
        d.wait()

  plsc.subcore_barrier()

  for k in range(8):
    c = sid * 8 + k
    @pl.when(c < NCH_N)
    def _():
      off = pl.multiple_of(c * K, 8)
      foff = pl.multiple_of(cid * N + c * K, 8)
      pltpu.sync_copy(degu_s.at[pl.ds(off, K)], line_v)
      pltpu.sync_copy(line_v, degu_h.at[pl.ds(foff, K)])
      pltpu.sync_copy(degv_s.at[pl.ds(off, K)], line_v)
      pltpu.sync_copy(line_v, degv_h.at[pl.ds(foff, K)])


@functools.partial(
    pl.kernel,
    out_type=(
        jax.ShapeDtypeStruct((NC, N, DH), _f32),
        jax.ShapeDtypeStruct((N,), _f32),
    ),
    mesh=_mesh,
    scratch_types=[
        pltpu.VMEM((SST[0][0], K2), jnp.int32),
        pltpu.VMEM((NCH_S, K2), jnp.int32),
        pltpu.VMEM((NB2, K2, DH), _f32),
        pltpu.VMEM((NB2, K2), _f32),
        pltpu.VMEM_SHARED((N, DH), _f32),
        pltpu.VMEM_SHARED((N,), _f32),
        pltpu.SemaphoreType.DMA,
        pltpu.SemaphoreType.DMA,
        pltpu.SemaphoreType.DMA,
        pltpu.SemaphoreType.DMA,
        pltpu.SemaphoreType.DMA,
        pltpu.SemaphoreType.DMA,
        pltpu.SemaphoreType.DMA,
    ],
    compiler_params=pltpu.CompilerParams(use_tc_tiling_on_sc=False),
)
def _sc_edge_agg(yl_h, yr_h, dinv_h, src_h, dst_h,
                 acc_h, s_h,
                 srcbuf, dstbuf, rbuf, vals_v,
                 acc_s, s_s,
                 semga, semgb, semsa, semsb, semsg, semsh, semss):
  cid = lax.axis_index("c")
  sid = lax.axis_index("s")

  @pl.loop(0, K2)
  def _(i):
    for j in range(DH // 16):
      rbuf[0, i, pl.ds(j * 16, 16)] = jnp.zeros((16,), _f32)

  for j in range(K2 // 16):
    vals_v[0, pl.ds(j * 16, 16)] = jnp.zeros((16,), _f32)

  for k in range(8):
    c = sid * 8 + k
    @pl.when(c < NCH_N2)
    def _():
      off = pl.multiple_of(c * K2, 8)
      pltpu.sync_copy(rbuf.at[0], acc_s.at[pl.ds(off, K2)])

      @pl.when(cid == 0)
      def _():
        pltpu.sync_copy(vals_v.at[0], s_s.at[pl.ds(off, K2)])

  pltpu.sync_copy(dst_h.at[sid], dstbuf)
  plsc.subcore_barrier()

  def _pipe(y_h, do_s):
    lo = 0
    for nch, nb in SST:
      H2 = nb // 2
      pltpu.sync_copy(src_h.at[sid, pl.ds(lo, nch)],
                      srcbuf.at[pl.ds(0, nch)])

      @pl.loop(0, nch // nb)
      def _(t):
        loc = t * nb
        base = lo + loc
        ga = [pltpu.async_copy(y_h.at[srcbuf.at[loc + b]], rbuf.at[b], semga)
              for b in range(H2)]
        gsa = [pltpu.async_copy(dinv_h.at[dstbuf.at[base + b]],
                                vals_v.at[b], semsg)
               for b in range(H2)] if do_s else []
        gb = [pltpu.async_copy(y_h.at[srcbuf.at[loc + H2 + b]],
                               rbuf.at[H2 + b], semgb)
              for b in range(H2)]
        gsb = [pltpu.async_copy(dinv_h.at[dstbuf.at[base + H2 + b]],
                                vals_v.at[H2 + b], semsh)
               for b in range(H2)] if do_s else []
        for d in ga:
          d.wait()
        sa = [pltpu.async_copy(rbuf.at[b], acc_s.at[dstbuf.at[base + b]],
                               semsa, add=True)
              for b in range(H2)]
        for d in gsa:
          d.wait()
        ssa = [pltpu.async_copy(vals_v.at[b],
                                s_s.at[srcbuf.at[loc + b]], semss, add=True)
               for b in range(H2)] if do_s else []
        for d in gb:
          d.wait()
        sb = [pltpu.async_copy(rbuf.at[H2 + b],
                               acc_s.at[dstbuf.at[base + H2 + b]],
                               semsb, add=True)
              for b in range(H2)]
        for d in gsb:
          d.wait()
        ssb = [pltpu.async_copy(vals_v.at[H2 + b],
                                s_s.at[srcbuf.at[loc + H2 + b]], semss,
                                add=True)
               for b in range(H2)] if do_s else []
        for d in sa + sb + ssa + ssb:
          d.wait()

      lo += nch

  @pl.when(cid == 0)
  def _():
    _pipe(yl_h, True)

  @pl.when(cid == 1)
  def _():
    _pipe(yr_h, False)

  plsc.subcore_barrier()

  for k in range(8):
    c = sid * 8 + k
    @pl.when(c < NCH_N2)
    def _():
      off = pl.multiple_of(c * K2, 8)
      pltpu.sync_copy(acc_s.at[pl.ds(off, K2)], rbuf.at[0])
      pltpu.sync_copy(rbuf.at[0], acc_h.at[cid, pl.ds(off, K2)])

      @pl.when(cid == 0)
      def _():
        pltpu.sync_copy(s_s.at[pl.ds(off, K2)], vals_v.at[0])
        pltpu.sync_copy(vals_v.at[0], s_h.at[pl.ds(off, K2)])


def _tc_prep(hu, hv, degu_t, degv_t, pW1, cW1):
  def body(hu_ref, hv_ref, degu_ref, degv_ref, wu_ref, wv_ref,
           ylu_ref, yru_ref, dinvu_ref, ylv_ref, yrv_ref, dinvv_ref):
    for h_ref, deg_ref, w_ref, yl_ref, yr_ref, dinv_ref in (
        (hu_ref, degu_ref, wu_ref, ylu_ref, yru_ref, dinvu_ref),
        (hv_ref, degv_ref, wv_ref, ylv_ref, yrv_ref, dinvv_ref)):
      deg = deg_ref[:, 0:1] + deg_ref[:, 1:2] + 1.0
      dinv = lax.rsqrt(jnp.maximum(deg, 1.0))
      dinv_ref[...] = dinv
      y = jnp.dot(dinv * h_ref[...], w_ref[...],
                  preferred_element_type=_f32,
                  precision=lax.Precision.HIGHEST)
      yl_ref[...] = y[:, :DH]
      yr_ref[...] = y[:, DH:]

  rd = pl.BlockSpec((RT, D), lambda i: (i, 0))
  rnc = pl.BlockSpec((RT, NC), lambda i: (i, 0))
  dd = pl.BlockSpec((D, D), lambda i: (0, 0))
  rh = pl.BlockSpec((RT, DH), lambda i: (i, 0))
  r1 = pl.BlockSpec((RT, 1), lambda i: (i, 0))
  return pl.pallas_call(
      body,
      grid=(GT,),
      in_specs=[rd, rd, rnc, rnc, dd, dd],
      out_specs=[rh, rh, r1, rh, rh, r1],
      out_shape=(
          jax.ShapeDtypeStruct((N, DH), _f32),
          jax.ShapeDtypeStruct((N, DH), _f32),
          jax.ShapeDtypeStruct((N, 1), _f32),
          jax.ShapeDtypeStruct((N, DH), _f32),
          jax.ShapeDtypeStruct((N, DH), _f32),
          jax.ShapeDtypeStruct((N, 1), _f32),
      ),
  )(hu, hv, degu_t, degv_t, pW1, cW1)


def _tc_readout(acc_u, ylu, yru, dinvu, su, b1u, W2u, b2u,
                acc_v, ylv, yrv, dinvv, sv, b1v, W2v, b2v,
                hq, Wm_u, Wm_v):
  def body(accu_ref, ylu_ref, yru_ref, dinvu_ref, su_ref,
           b1u_ref, w2u_ref, b2u_ref,
           accv_ref, ylv_ref, yrv_ref, dinvv_ref, sv_ref,
           b1v_ref, w2v_ref, b2v_ref,
           hq_ref, wmu_ref, wmv_ref, out_ref, t_ref):
    i = pl.program_id(0)

    @pl.when(i == 0)
    def _():
      t_ref[...] = jnp.zeros_like(t_ref)

    for g, (acc_ref, yl_ref, yr_ref, dinv_ref, s_ref, b1_ref) in enumerate((
        (accu_ref, ylu_ref, yru_ref, dinvu_ref, su_ref, b1u_ref),
        (accv_ref, ylv_ref, yrv_ref, dinvv_ref, sv_ref, b1v_ref))):
      dinv = dinv_ref[...]
      w = dinv * (dinv + s_ref[...]) * (1.0 / N)
      for half, acc_half, y_ref in ((0, acc_ref[0], yl_ref),
                                    (1, acc_ref[1], yr_ref)):
        acc = acc_half + y_ref[...]
        pre = dinv * acc + b1_ref[:, half * DH:(half + 1) * DH]
        h1 = jnp.where(pre >= 0, pre, 0.01 * pre)
        t_ref[g:g + 1, half * DH:(half + 1) * DH] += jnp.sum(
            w * h1, axis=0, keepdims=True)

    @pl.when(i == GT - 1)
    def _():
      hk = dict(preferred_element_type=_f32, precision=lax.Precision.HIGHEST)
      dn = (((1,), (1,)), ((), ()))
      hgu = jnp.dot(t_ref[0:1], w2u_ref[...], **hk) + b2u_ref[...]
      hgv = jnp.dot(t_ref[1:2], w2v_ref[...], **hk) + b2v_ref[...]
      z = (lax.dot_general(hgu, wmu_ref[...], dn, **hk) +
           lax.dot_general(hgv, wmv_ref[...], dn, **hk))
      out_ref[...] = lax.dot_general(hq_ref[...], z, dn, **hk)

  acc_s_ = pl.BlockSpec((NC, RT, DH), lambda i: (0, i, 0))
  rh = pl.BlockSpec((RT, DH), lambda i: (i, 0))
  r1 = pl.BlockSpec((RT, 1), lambda i: (i, 0))
  od = pl.BlockSpec((1, D), lambda i: (0, 0))
  dd = pl.BlockSpec((D, D), lambda i: (0, 0))
  bd = pl.BlockSpec((B, D), lambda i: (0, 0))
  b1_ = pl.BlockSpec((B, 1), lambda i: (0, 0))
  return pl.pallas_call(
      body,
      grid=(GT,),
      in_specs=[acc_s_, rh, rh, r1, r1, od, dd, od,
                acc_s_, rh, rh, r1, r1, od, dd, od,
                bd, dd, dd],
      out_specs=b1_,
      out_shape=jax.ShapeDtypeStruct((B, 1), _f32),
      scratch_shapes=[pltpu.VMEM((2, D), _f32)],
  )(acc_u, ylu, yru, dinvu, su, b1u, W2u, b2u,
    acc_v, ylv, yrv, dinvv, sv, b1v, W2v, b2v,
    hq, Wm_u, Wm_v)


def kernel(u_node_ids, u_edge_index, v_node_ids, v_edge_index, q,
           embedding, pW1, pb1, pW2, pb2, cW1, cb1, cW2, cb2, Wm_u, Wm_v):
  u_ids = u_node_ids.astype(jnp.int32)
  v_ids = v_node_ids.astype(jnp.int32)
  qi = q.astype(jnp.int32)
  ue = u_edge_index.astype(jnp.int32)
  ve = v_edge_index.astype(jnp.int32)
  dstw_u = ue[1].reshape(NW, NCH_E, K)
  dstw_v = ve[1].reshape(NW, NCH_E, K)
  src_u = ue[0].reshape(NS, NCH_S, K2)
  dst_u = ue[1].reshape(NS, NCH_S, K2)
  src_v = ve[0].reshape(NS, NCH_S, K2)
  dst_v = ve[1].reshape(NS, NCH_S, K2)

  hu, hv, hq, degu_p, degv_p = _sc_gather_deg(
      u_ids, v_ids, qi, embedding, dstw_u, dstw_v)

  yl_u, yr_u, dinv_u, yl_v, yr_v, dinv_v = _tc_prep(
      hu, hv, degu_p.reshape(NC, N).T, degv_p.reshape(NC, N).T, pW1, cW1)

  acc_u_p, s_u = _sc_edge_agg(yl_u, yr_u, dinv_u.reshape(N), src_u, dst_u)
  acc_v_p, s_v = _sc_edge_agg(yl_v, yr_v, dinv_v.reshape(N), src_v, dst_v)

  scores = _tc_readout(
      acc_u_p, yl_u, yr_u, dinv_u, s_u.reshape(N, 1),
      pb1.reshape(1, D), pW2, pb2.reshape(1, D),
      acc_v_p, yl_v, yr_v, dinv_v, s_v.reshape(N, 1),
      cb1.reshape(1, D), cW2, cb2.reshape(1, D),
      hq, Wm_u, Wm_v)
  return scores.reshape(B)
